# baseline (speedup 1.0000x reference)
import functools

import jax
import jax.numpy as jnp
from jax import lax
from jax.experimental import pallas as pl
from jax.experimental.pallas import tpu as pltpu
from jax.experimental.pallas import tpu_sc as plsc

N = 1048576
K = 16384
CAP = 128
NC, NS = 2, 16
NW = NC * NS
CHUNK = N // NW
PIECE = 2048
SEGCAP = 1280
BPW = K // NW
BPH = BPW // 2
VECS_B = SEGCAP // 16

def _wid():
    return lax.axis_index("s") * NC + lax.axis_index("c")


def _sc_part_body(t_hbm, r_hbm, e_hbm, stag_t, stag_sx, cnt_out, er_out, e_out,
                  stg_t, stg_sx, cnt32, tbuf, rbuf, ebuf, erbuf, ebuf16, insem):
    wid = _wid()
    base = wid * CHUNK

    def zbody(i, carry):
        cnt32[pl.ds(i * 16, 16)] = jnp.zeros((16,), jnp.int32)
        return carry

    lax.fori_loop(0, 2, zbody, 0)

    def piece_body(p, accs):
        off = base + p * PIECE
        din = [
            pltpu.async_copy(t_hbm.at[pl.ds(off, PIECE)], tbuf, insem),
            pltpu.async_copy(r_hbm.at[pl.ds(off, PIECE)], rbuf, insem),
            pltpu.async_copy(e_hbm.at[pl.ds(off, PIECE)], ebuf, insem),
        ]
        for d in din:
            d.wait()

        def vec_body(v, accs2):
            er_a, e_a = accs2
            t = tbuf[pl.ds(v * 16, 16)]
            r = rbuf[pl.ds(v * 16, 16)]
            e = ebuf[pl.ds(v * 16, 16)]
            b = jnp.minimum((t * float(K)).astype(jnp.int32), K - 1)
            dst = lax.shift_right_logical(b, 9)
            old = plsc.load_gather(cnt32, [dst])
            dupc, last = plsc.scan_count(dst)
            plsc.addupdate_scatter(cnt32, [dst], dupc, mask=last)
            pos = jnp.minimum(old + dupc - 1, SEGCAP - 1)
            ex = jnp.exp(r)
            sx = jnp.where(e > 0.0, ex, -ex)
            plsc.store_scatter(stg_t, [dst, pos], t)
            plsc.store_scatter(stg_sx, [dst, pos], sx)
            return (er_a + e * r, e_a + e)

        return lax.fori_loop(0, PIECE // 16, vec_body, accs)

    zero16f = jnp.zeros((16,), jnp.float32)
    er_acc, e_acc = lax.fori_loop(0, CHUNK // PIECE, piece_body,
                                  (zero16f, zero16f))
    erbuf[...] = er_acc
    ebuf16[...] = e_acc

    descs = []
    for d in range(NW):
        descs.append(pltpu.async_copy(stg_t.at[d], stag_t.at[d, wid], insem))
        descs.append(pltpu.async_copy(stg_sx.at[d], stag_sx.at[d, wid], insem))
    for dd in descs:
        dd.wait()
    pltpu.sync_copy(cnt32, cnt_out.at[wid])
    pltpu.sync_copy(erbuf, er_out.at[wid])
    pltpu.sync_copy(ebuf16, e_out.at[wid])


def _sc_group_body(stag_t, stag_sx, cnt_hbm, ts_out, sx_out, nb_out,
                   ts_tab, sx_tab, ctab, cnt_vm, tb, sb, insem):
    wid = _wid()
    pltpu.sync_copy(cnt_hbm, cnt_vm)

    for h in range(2):
        hbase = wid * BPW + h * BPH

        def zbody(i, carry):
            ctab[pl.ds(i * 16, 16)] = jnp.zeros((16,), jnp.int32)
            return carry

        lax.fori_loop(0, BPH // 16, zbody, 0)

        def src_body(src, carry):
            din = [
                pltpu.async_copy(stag_t.at[wid, src], tb, insem),
                pltpu.async_copy(stag_sx.at[wid, src], sb, insem),
            ]
            for d in din:
                d.wait()
            srcv = jnp.zeros((16,), jnp.int32) + src
            widv = jnp.zeros((16,), jnp.int32) + wid
            cntv = plsc.load_gather(cnt_vm, [srcv, widv])
            cntv = jnp.minimum(cntv, SEGCAP)

            def vec_body(v, carry2):
                t = tb[pl.ds(v * 16, 16)]
                sx = sb[pl.ds(v * 16, 16)]
                gpos = v * 16 + lax.iota(jnp.int32, 16)
                gmask = gpos < cntv
                b = jnp.minimum((t * float(K)).astype(jnp.int32), K - 1)
                bl = b - hbase
                inb = gmask & (bl >= 0) & (bl < BPH)
                blc = jnp.clip(bl, 0, BPH - 1)
                old = plsc.load_gather(ctab, [blc])
                dupc, last = plsc.scan_count(blc, mask=inb)
                plsc.addupdate_scatter(ctab, [blc], dupc, mask=last & inb)
                slot = jnp.minimum(old + dupc - 1, CAP - 1)
                flat = blc * CAP + slot
                plsc.store_scatter(ts_tab, [flat], t, mask=inb)
                plsc.store_scatter(sx_tab, [flat], sx, mask=inb)
                return carry2

            return lax.fori_loop(0, VECS_B, vec_body, carry)

        lax.fori_loop(0, NW, src_body, 0)
        obase = hbase * CAP
        dout = [
            pltpu.async_copy(ts_tab, ts_out.at[pl.ds(obase, BPH * CAP)], insem),
            pltpu.async_copy(sx_tab, sx_out.at[pl.ds(obase, BPH * CAP)], insem),
            pltpu.async_copy(ctab, nb_out.at[pl.ds(hbase, BPH)], insem),
        ]
        for d in dout:
            d.wait()


NB = 64


def _tc_main_body(ts_ref, sx_ref, n_ref, erp_ref, ep_ref, out_ref, acc):
    g = pl.program_id(0)
    nsteps = pl.num_programs(0)

    @pl.when(g == 0)
    def _():
        acc[0] = 0.0
        acc[1] = 0.0
        acc[2] = 0.0

    ts = ts_ref[...]
    sx = sx_ref[...]
    n = n_ref[...]
    lanes = lax.broadcasted_iota(jnp.int32, (NB, CAP), 1)
    valid = lanes < jnp.minimum(n, CAP)
    expr = jnp.where(valid, jnp.abs(sx), 0.0)

    srow = jnp.sum(expr, axis=1, keepdims=True)
    ri = lax.broadcasted_iota(jnp.int32, (NB, NB), 0)
    rj = lax.broadcasted_iota(jnp.int32, (NB, NB), 1)
    tri = (rj > ri).astype(jnp.float32)
    suf = jax.lax.dot_general(tri, srow, (((1,), (0,)), ((), ())),
                              preferred_element_type=jnp.float32)
    t_base = acc[2] + suf

    k = lax.bitcast_convert_type(ts, jnp.int32)
    sj = (2 * k + 1)[:, None, :]
    ioi = lax.broadcasted_iota(jnp.int32, (CAP, CAP), 0)
    ioj = lax.broadcasted_iota(jnp.int32, (CAP, CAP), 1)
    jgti = (ioj > ioi).astype(jnp.int32)[None, :, :]
    si = (2 * k)[:, :, None] + jgti
    maskf = (sj > si).astype(jnp.float32)
    w = lax.dot_general(maskf, expr, (((2,), (1,)), ((0,), (0,))),
                        preferred_element_type=jnp.float32)

    c = t_base + w
    e_on = valid & (sx > 0.0)
    contrib = jnp.sum(jnp.where(e_on, jnp.log(c), 0.0))

    y = contrib - acc[1]
    t_new = acc[0] + y
    acc[1] = (t_new - acc[0]) - y
    acc[0] = t_new
    acc[2] = acc[2] + jnp.sum(srow)

    @pl.when(g == nsteps - 1)
    def _():
        er_tot = jnp.sum(erp_ref[...])
        e_tot = jnp.sum(ep_ref[...])
        out_ref[...] = jnp.full((1, 1), (acc[0] - er_tot) / e_tot,
                                dtype=jnp.float32)


_tc_main = pl.pallas_call(
    _tc_main_body,
    grid=(K // NB,),
    in_specs=[
        pl.BlockSpec((NB, CAP), lambda g: (K // NB - 1 - g, 0)),
        pl.BlockSpec((NB, CAP), lambda g: (K // NB - 1 - g, 0)),
        pl.BlockSpec((NB, 1), lambda g: (K // NB - 1 - g, 0)),
        pl.BlockSpec((NW, 16), lambda g: (0, 0)),
        pl.BlockSpec((NW, 16), lambda g: (0, 0)),
    ],
    out_specs=pl.BlockSpec((1, 1), lambda g: (0, 0)),
    out_shape=jax.ShapeDtypeStruct((1, 1), jnp.float32),
    scratch_shapes=[pltpu.SMEM((3,), jnp.float32)],
)


@functools.lru_cache(maxsize=1)
def _build_sc_kernels():
    mesh = plsc.VectorSubcoreMesh(
        core_axis_name="c", subcore_axis_name="s", num_cores=NC, num_subcores=NS
    )
    sc_params = pltpu.CompilerParams(needs_layout_passes=False)
    sc_part = pl.kernel(
        _sc_part_body,
        out_type=(
            jax.ShapeDtypeStruct((NW, NW, SEGCAP), jnp.float32),
            jax.ShapeDtypeStruct((NW, NW, SEGCAP), jnp.float32),
            jax.ShapeDtypeStruct((NW, NW), jnp.int32),
            jax.ShapeDtypeStruct((NW, 16), jnp.float32),
            jax.ShapeDtypeStruct((NW, 16), jnp.float32),
        ),
        mesh=mesh,
        scratch_types=[
            pltpu.VMEM((NW, SEGCAP), jnp.float32),
            pltpu.VMEM((NW, SEGCAP), jnp.float32),
            pltpu.VMEM((NW,), jnp.int32),
            pltpu.VMEM((PIECE,), jnp.float32),
            pltpu.VMEM((PIECE,), jnp.float32),
            pltpu.VMEM((PIECE,), jnp.float32),
            pltpu.VMEM((16,), jnp.float32),
            pltpu.VMEM((16,), jnp.float32),
            pltpu.SemaphoreType.DMA,
        ],
        compiler_params=sc_params,
    )
    sc_group = pl.kernel(
        _sc_group_body,
        out_type=(
            jax.ShapeDtypeStruct((K * CAP,), jnp.float32),
            jax.ShapeDtypeStruct((K * CAP,), jnp.float32),
            jax.ShapeDtypeStruct((K,), jnp.int32),
        ),
        mesh=mesh,
        scratch_types=[
            pltpu.VMEM((BPH * CAP,), jnp.float32),
            pltpu.VMEM((BPH * CAP,), jnp.float32),
            pltpu.VMEM((BPH,), jnp.int32),
            pltpu.VMEM((NW, NW), jnp.int32),
            pltpu.VMEM((SEGCAP,), jnp.float32),
            pltpu.VMEM((SEGCAP,), jnp.float32),
            pltpu.SemaphoreType.DMA,
        ],
        compiler_params=sc_params,
    )
    return sc_part, sc_group


def kernel(risk_scores, times, events):
    sc_part, sc_group = _build_sc_kernels()
    stag_t, stag_sx, cnts, er_p, e_p = sc_part(times, risk_scores, events)
    ts_flat, sx_flat, nb = sc_group(stag_t, stag_sx, cnts)
    ts2 = ts_flat.reshape(K, CAP)
    sx2 = sx_flat.reshape(K, CAP)
    ncol = nb.reshape(K, 1)
    out = _tc_main(ts2, sx2, ncol, er_p, e_p)
    return out[0, 0]


# speedup vs baseline: 1.2116x; 1.2116x over previous
import functools

import jax
import jax.numpy as jnp
from jax import lax
from jax.experimental import pallas as pl
from jax.experimental.pallas import tpu as pltpu
from jax.experimental.pallas import tpu_sc as plsc

N = 1048576
K = 16384
CAP = 128
NC, NS = 2, 16
NW = NC * NS
CHUNK = N // NW
PIECE = 2048
SEGCAP = 1280
BPW = K // NW
BPH = BPW // 2
VECS_B = SEGCAP // 16

def _wid():
    return lax.axis_index("s") * NC + lax.axis_index("c")


def _sc_part_body(t_hbm, r_hbm, e_hbm, stag_t, stag_sx, cnt_out, er_out, e_out,
                  stg_t, stg_sx, cnt32, tbuf, rbuf, ebuf, erbuf, ebuf16, insem):
    wid = _wid()
    base = wid * CHUNK

    def zbody(i, carry):
        cnt32[pl.ds(i * 16, 16)] = jnp.zeros((16,), jnp.int32)
        return carry

    lax.fori_loop(0, 2, zbody, 0)

    def piece_body(p, accs):
        off = base + p * PIECE
        din = [
            pltpu.async_copy(t_hbm.at[pl.ds(off, PIECE)], tbuf, insem),
            pltpu.async_copy(r_hbm.at[pl.ds(off, PIECE)], rbuf, insem),
            pltpu.async_copy(e_hbm.at[pl.ds(off, PIECE)], ebuf, insem),
        ]
        for d in din:
            d.wait()

        def vec_body(v, accs2):
            er_a, e_a = accs2
            t = tbuf[pl.ds(v * 16, 16)]
            r = rbuf[pl.ds(v * 16, 16)]
            e = ebuf[pl.ds(v * 16, 16)]
            b = jnp.minimum((t * float(K)).astype(jnp.int32), K - 1)
            dst = lax.shift_right_logical(b, 9)
            old = plsc.load_gather(cnt32, [dst])
            dupc, last = plsc.scan_count(dst)
            plsc.addupdate_scatter(cnt32, [dst], dupc, mask=last)
            pos = jnp.minimum(old + dupc - 1, SEGCAP - 1)
            ex = jnp.exp(r)
            sx = jnp.where(e > 0.0, ex, -ex)
            plsc.store_scatter(stg_t, [dst, pos], t)
            plsc.store_scatter(stg_sx, [dst, pos], sx)
            return (er_a + e * r, e_a + e)

        return lax.fori_loop(0, PIECE // 16, vec_body, accs)

    zero16f = jnp.zeros((16,), jnp.float32)
    er_acc, e_acc = lax.fori_loop(0, CHUNK // PIECE, piece_body,
                                  (zero16f, zero16f))
    erbuf[...] = er_acc
    ebuf16[...] = e_acc

    descs = []
    for d in range(NW):
        descs.append(pltpu.async_copy(stg_t.at[d], stag_t.at[d, wid], insem))
        descs.append(pltpu.async_copy(stg_sx.at[d], stag_sx.at[d, wid], insem))
    for dd in descs:
        dd.wait()
    pltpu.sync_copy(cnt32, cnt_out.at[wid])
    pltpu.sync_copy(erbuf, er_out.at[wid])
    pltpu.sync_copy(ebuf16, e_out.at[wid])


def _sc_group_body(stag_t, stag_sx, cnt_hbm, ts_out, sx_out, nb_out,
                   ts_tab, sx_tab, ctab, cnt_vm, tb, sb, insem):
    wid = _wid()
    pltpu.sync_copy(cnt_hbm, cnt_vm)

    for h in range(2):
        hbase = wid * BPW + h * BPH

        def zbody(i, carry):
            ctab[pl.ds(i * 16, 16)] = jnp.zeros((16,), jnp.int32)
            return carry

        lax.fori_loop(0, BPH // 16, zbody, 0)

        def src_body(src, carry):
            din = [
                pltpu.async_copy(stag_t.at[wid, src], tb, insem),
                pltpu.async_copy(stag_sx.at[wid, src], sb, insem),
            ]
            for d in din:
                d.wait()
            srcv = jnp.zeros((16,), jnp.int32) + src
            widv = jnp.zeros((16,), jnp.int32) + wid
            cntv = plsc.load_gather(cnt_vm, [srcv, widv])
            cntv = jnp.minimum(cntv, SEGCAP)

            def vec_body(v, carry2):
                t = tb[pl.ds(v * 16, 16)]
                sx = sb[pl.ds(v * 16, 16)]
                gpos = v * 16 + lax.iota(jnp.int32, 16)
                gmask = gpos < cntv
                b = jnp.minimum((t * float(K)).astype(jnp.int32), K - 1)
                bl = b - hbase
                inb = gmask & (bl >= 0) & (bl < BPH)
                blc = jnp.clip(bl, 0, BPH - 1)
                old = plsc.load_gather(ctab, [blc])
                dupc, last = plsc.scan_count(blc, mask=inb)
                plsc.addupdate_scatter(ctab, [blc], dupc, mask=last & inb)
                slot = jnp.minimum(old + dupc - 1, CAP - 1)
                flat = blc * CAP + slot
                plsc.store_scatter(ts_tab, [flat], t, mask=inb)
                plsc.store_scatter(sx_tab, [flat], sx, mask=inb)
                return carry2

            return lax.fori_loop(0, VECS_B, vec_body, carry)

        lax.fori_loop(0, NW, src_body, 0)
        obase = hbase * CAP
        dout = [
            pltpu.async_copy(ts_tab, ts_out.at[pl.ds(obase, BPH * CAP)], insem),
            pltpu.async_copy(sx_tab, sx_out.at[pl.ds(obase, BPH * CAP)], insem),
            pltpu.async_copy(ctab, nb_out.at[pl.ds(hbase, BPH)], insem),
        ]
        for d in dout:
            d.wait()


NB = 32


def _tc_main_body(ts_ref, sx_ref, n_ref, erp_ref, ep_ref, out_ref, acc):
    g = pl.program_id(0)
    nsteps = pl.num_programs(0)

    @pl.when(g == 0)
    def _():
        acc[0] = 0.0
        acc[1] = 0.0
        acc[2] = 0.0

    ts = ts_ref[...]
    sx = sx_ref[...]
    n = n_ref[...]
    lanes = lax.broadcasted_iota(jnp.int32, (NB, CAP), 1)
    valid = lanes < jnp.minimum(n, CAP)
    expr = jnp.where(valid, jnp.abs(sx), 0.0)

    srow = jnp.sum(expr, axis=1, keepdims=True)
    ri = lax.broadcasted_iota(jnp.int32, (NB, NB), 0)
    rj = lax.broadcasted_iota(jnp.int32, (NB, NB), 1)
    tri = (rj > ri).astype(jnp.float32)
    suf = jax.lax.dot_general(tri, srow, (((1,), (0,)), ((), ())),
                              preferred_element_type=jnp.float32)
    t_base = acc[2] + suf

    k = lax.bitcast_convert_type(ts, jnp.int32)
    sj = (2 * k + 1)[:, None, :]
    ioi = lax.broadcasted_iota(jnp.int32, (CAP, CAP), 0)
    ioj = lax.broadcasted_iota(jnp.int32, (CAP, CAP), 1)
    jgti = (ioj > ioi).astype(jnp.int32)[None, :, :]
    si = (2 * k)[:, :, None] + jgti
    mask = sj > si
    w = jnp.sum(jnp.where(mask, expr[:, None, :], 0.0), axis=2)

    c = t_base + w
    e_on = valid & (sx > 0.0)
    contrib = jnp.sum(jnp.where(e_on, jnp.log(c), 0.0))

    y = contrib - acc[1]
    t_new = acc[0] + y
    acc[1] = (t_new - acc[0]) - y
    acc[0] = t_new
    acc[2] = acc[2] + jnp.sum(srow)

    @pl.when(g == nsteps - 1)
    def _():
        er_tot = jnp.sum(erp_ref[...])
        e_tot = jnp.sum(ep_ref[...])
        out_ref[...] = jnp.full((1, 1), (acc[0] - er_tot) / e_tot,
                                dtype=jnp.float32)


_tc_main = pl.pallas_call(
    _tc_main_body,
    grid=(K // NB,),
    in_specs=[
        pl.BlockSpec((NB, CAP), lambda g: (K // NB - 1 - g, 0)),
        pl.BlockSpec((NB, CAP), lambda g: (K // NB - 1 - g, 0)),
        pl.BlockSpec((NB, 1), lambda g: (K // NB - 1 - g, 0)),
        pl.BlockSpec((NW, 16), lambda g: (0, 0)),
        pl.BlockSpec((NW, 16), lambda g: (0, 0)),
    ],
    out_specs=pl.BlockSpec((1, 1), lambda g: (0, 0)),
    out_shape=jax.ShapeDtypeStruct((1, 1), jnp.float32),
    scratch_shapes=[pltpu.SMEM((3,), jnp.float32)],
)


@functools.lru_cache(maxsize=1)
def _build_sc_kernels():
    mesh = plsc.VectorSubcoreMesh(
        core_axis_name="c", subcore_axis_name="s", num_cores=NC, num_subcores=NS
    )
    sc_params = pltpu.CompilerParams(needs_layout_passes=False)
    sc_part = pl.kernel(
        _sc_part_body,
        out_type=(
            jax.ShapeDtypeStruct((NW, NW, SEGCAP), jnp.float32),
            jax.ShapeDtypeStruct((NW, NW, SEGCAP), jnp.float32),
            jax.ShapeDtypeStruct((NW, NW), jnp.int32),
            jax.ShapeDtypeStruct((NW, 16), jnp.float32),
            jax.ShapeDtypeStruct((NW, 16), jnp.float32),
        ),
        mesh=mesh,
        scratch_types=[
            pltpu.VMEM((NW, SEGCAP), jnp.float32),
            pltpu.VMEM((NW, SEGCAP), jnp.float32),
            pltpu.VMEM((NW,), jnp.int32),
            pltpu.VMEM((PIECE,), jnp.float32),
            pltpu.VMEM((PIECE,), jnp.float32),
            pltpu.VMEM((PIECE,), jnp.float32),
            pltpu.VMEM((16,), jnp.float32),
            pltpu.VMEM((16,), jnp.float32),
            pltpu.SemaphoreType.DMA,
        ],
        compiler_params=sc_params,
    )
    sc_group = pl.kernel(
        _sc_group_body,
        out_type=(
            jax.ShapeDtypeStruct((K * CAP,), jnp.float32),
            jax.ShapeDtypeStruct((K * CAP,), jnp.float32),
            jax.ShapeDtypeStruct((K,), jnp.int32),
        ),
        mesh=mesh,
        scratch_types=[
            pltpu.VMEM((BPH * CAP,), jnp.float32),
            pltpu.VMEM((BPH * CAP,), jnp.float32),
            pltpu.VMEM((BPH,), jnp.int32),
            pltpu.VMEM((NW, NW), jnp.int32),
            pltpu.VMEM((SEGCAP,), jnp.float32),
            pltpu.VMEM((SEGCAP,), jnp.float32),
            pltpu.SemaphoreType.DMA,
        ],
        compiler_params=sc_params,
    )
    return sc_part, sc_group


def kernel(risk_scores, times, events):
    sc_part, sc_group = _build_sc_kernels()
    stag_t, stag_sx, cnts, er_p, e_p = sc_part(times, risk_scores, events)
    ts_flat, sx_flat, nb = sc_group(stag_t, stag_sx, cnts)
    ts2 = ts_flat.reshape(K, CAP)
    sx2 = sx_flat.reshape(K, CAP)
    ncol = nb.reshape(K, 1)
    out = _tc_main(ts2, sx2, ncol, er_p, e_p)
    return out[0, 0]


# speedup vs baseline: 1.3519x; 1.1158x over previous
import functools

import jax
import jax.numpy as jnp
from jax import lax
from jax.experimental import pallas as pl
from jax.experimental.pallas import tpu as pltpu
from jax.experimental.pallas import tpu_sc as plsc

N = 1048576
K = 16384
CAP = 128
NC, NS = 2, 16
NW = NC * NS
CHUNK = N // NW
PIECE = 2048
SEGCAP = 1280
BPW = K // NW
BPH = BPW // 2
VECS_B = SEGCAP // 16

def _wid():
    return lax.axis_index("s") * NC + lax.axis_index("c")


def _sc_part_body(t_hbm, r_hbm, e_hbm, stag_t, stag_sx, cnt_out, er_out, e_out,
                  stg_t, stg_sx, cnt32, tbuf, rbuf, ebuf, erbuf, ebuf16, insem):
    wid = _wid()
    base = wid * CHUNK

    def zbody(i, carry):
        cnt32[pl.ds(i * 16, 16)] = jnp.zeros((16,), jnp.int32)
        return carry

    lax.fori_loop(0, 2, zbody, 0)

    def piece_body(p, accs):
        off = base + p * PIECE
        din = [
            pltpu.async_copy(t_hbm.at[pl.ds(off, PIECE)], tbuf, insem),
            pltpu.async_copy(r_hbm.at[pl.ds(off, PIECE)], rbuf, insem),
            pltpu.async_copy(e_hbm.at[pl.ds(off, PIECE)], ebuf, insem),
        ]
        for d in din:
            d.wait()

        def vec_body(v, accs2):
            er_a, e_a = accs2
            t = tbuf[pl.ds(v * 16, 16)]
            r = rbuf[pl.ds(v * 16, 16)]
            e = ebuf[pl.ds(v * 16, 16)]
            b = jnp.minimum((t * float(K)).astype(jnp.int32), K - 1)
            dst = lax.shift_right_logical(b, 9)
            old = plsc.load_gather(cnt32, [dst])
            dupc, last = plsc.scan_count(dst)
            plsc.addupdate_scatter(cnt32, [dst], dupc, mask=last)
            pos = jnp.minimum(old + dupc - 1, SEGCAP - 1)
            ex = jnp.exp(r)
            sx = jnp.where(e > 0.0, ex, -ex)
            plsc.store_scatter(stg_t, [dst, pos], t)
            plsc.store_scatter(stg_sx, [dst, pos], sx)
            return (er_a + e * r, e_a + e)

        return lax.fori_loop(0, PIECE // 16, vec_body, accs)

    zero16f = jnp.zeros((16,), jnp.float32)
    er_acc, e_acc = lax.fori_loop(0, CHUNK // PIECE, piece_body,
                                  (zero16f, zero16f))
    erbuf[...] = er_acc
    ebuf16[...] = e_acc

    descs = []
    for d in range(NW):
        descs.append(pltpu.async_copy(stg_t.at[d], stag_t.at[d, wid], insem))
        descs.append(pltpu.async_copy(stg_sx.at[d], stag_sx.at[d, wid], insem))
    for dd in descs:
        dd.wait()
    pltpu.sync_copy(cnt32, cnt_out.at[wid])
    pltpu.sync_copy(erbuf, er_out.at[wid])
    pltpu.sync_copy(ebuf16, e_out.at[wid])


def _sc_group_body(stag_t, stag_sx, cnt_hbm, ts_out, sx_out, nb_out,
                   ts_tab, sx_tab, ctab, cnt_vm, tb, sb, insem):
    wid = _wid()
    pltpu.sync_copy(cnt_hbm, cnt_vm)

    for h in range(2):
        hbase = wid * BPW + h * BPH

        def zbody(i, carry):
            ctab[pl.ds(i * 16, 16)] = jnp.zeros((16,), jnp.int32)
            return carry

        lax.fori_loop(0, BPH // 16, zbody, 0)

        def src_body(src, carry):
            din = [
                pltpu.async_copy(stag_t.at[wid, src], tb, insem),
                pltpu.async_copy(stag_sx.at[wid, src], sb, insem),
            ]
            for d in din:
                d.wait()
            srcv = jnp.zeros((16,), jnp.int32) + src
            widv = jnp.zeros((16,), jnp.int32) + wid
            cntv = plsc.load_gather(cnt_vm, [srcv, widv])
            cntv = jnp.minimum(cntv, SEGCAP)

            def vec_body(v, carry2):
                t = tb[pl.ds(v * 16, 16)]
                sx = sb[pl.ds(v * 16, 16)]
                gpos = v * 16 + lax.iota(jnp.int32, 16)
                gmask = gpos < cntv
                b = jnp.minimum((t * float(K)).astype(jnp.int32), K - 1)
                bl = b - hbase
                inb = gmask & (bl >= 0) & (bl < BPH)
                blc = jnp.clip(bl, 0, BPH - 1)
                old = plsc.load_gather(ctab, [blc])
                dupc, last = plsc.scan_count(blc, mask=inb)
                plsc.addupdate_scatter(ctab, [blc], dupc, mask=last & inb)
                slot = jnp.minimum(old + dupc - 1, CAP - 1)
                flat = blc * CAP + slot
                plsc.store_scatter(ts_tab, [flat], t, mask=inb)
                plsc.store_scatter(sx_tab, [flat], sx, mask=inb)
                return carry2

            return lax.fori_loop(0, VECS_B, vec_body, carry)

        lax.fori_loop(0, NW, src_body, 0)
        obase = hbase * CAP
        dout = [
            pltpu.async_copy(ts_tab, ts_out.at[pl.ds(obase, BPH * CAP)], insem),
            pltpu.async_copy(sx_tab, sx_out.at[pl.ds(obase, BPH * CAP)], insem),
            pltpu.async_copy(ctab, nb_out.at[pl.ds(hbase, BPH)], insem),
        ]
        for d in dout:
            d.wait()


NB = 128


def _tc_main_body(ts_ref, sx_ref, n_ref, erp_ref, ep_ref, out_ref, acc):
    g = pl.program_id(0)
    nsteps = pl.num_programs(0)

    @pl.when(g == 0)
    def _():
        acc[0] = 0.0
        acc[1] = 0.0
        acc[2] = 0.0

    ts = ts_ref[...]
    sx = sx_ref[...]
    n = n_ref[...]
    lanes = lax.broadcasted_iota(jnp.int32, (NB, CAP), 1)
    valid = lanes < jnp.minimum(n, CAP)
    expr = jnp.where(valid, jnp.abs(sx), 0.0)

    srow = jnp.sum(expr, axis=1, keepdims=True)
    ri = lax.broadcasted_iota(jnp.int32, (NB, NB), 0)
    rj = lax.broadcasted_iota(jnp.int32, (NB, NB), 1)
    tri = (rj > ri).astype(jnp.float32)
    suf = jax.lax.dot_general(tri, srow, (((1,), (0,)), ((), ())),
                              preferred_element_type=jnp.float32)
    t_base = acc[2] + suf

    k = lax.bitcast_convert_type(ts, jnp.int32)
    sj = (2 * k + 1)[:, None, :]
    ioi = lax.broadcasted_iota(jnp.int32, (CAP, CAP), 0)
    ioj = lax.broadcasted_iota(jnp.int32, (CAP, CAP), 1)
    jgti = (ioj > ioi).astype(jnp.int32)[None, :, :]
    si = (2 * k)[:, :, None] + jgti
    mask = sj > si
    w = jnp.sum(jnp.where(mask, expr[:, None, :], 0.0), axis=2)

    c = t_base + w
    e_on = valid & (sx > 0.0)
    contrib = jnp.sum(jnp.where(e_on, jnp.log(c), 0.0))

    y = contrib - acc[1]
    t_new = acc[0] + y
    acc[1] = (t_new - acc[0]) - y
    acc[0] = t_new
    acc[2] = acc[2] + jnp.sum(srow)

    @pl.when(g == nsteps - 1)
    def _():
        er_tot = jnp.sum(erp_ref[...])
        e_tot = jnp.sum(ep_ref[...])
        out_ref[...] = jnp.full((1, 1), (acc[0] - er_tot) / e_tot,
                                dtype=jnp.float32)


_tc_main = pl.pallas_call(
    _tc_main_body,
    grid=(K // NB,),
    in_specs=[
        pl.BlockSpec((NB, CAP), lambda g: (K // NB - 1 - g, 0)),
        pl.BlockSpec((NB, CAP), lambda g: (K // NB - 1 - g, 0)),
        pl.BlockSpec((NB, 1), lambda g: (K // NB - 1 - g, 0)),
        pl.BlockSpec((NW, 16), lambda g: (0, 0)),
        pl.BlockSpec((NW, 16), lambda g: (0, 0)),
    ],
    out_specs=pl.BlockSpec((1, 1), lambda g: (0, 0)),
    out_shape=jax.ShapeDtypeStruct((1, 1), jnp.float32),
    scratch_shapes=[pltpu.SMEM((3,), jnp.float32)],
)


@functools.lru_cache(maxsize=1)
def _build_sc_kernels():
    mesh = plsc.VectorSubcoreMesh(
        core_axis_name="c", subcore_axis_name="s", num_cores=NC, num_subcores=NS
    )
    sc_params = pltpu.CompilerParams(needs_layout_passes=False)
    sc_part = pl.kernel(
        _sc_part_body,
        out_type=(
            jax.ShapeDtypeStruct((NW, NW, SEGCAP), jnp.float32),
            jax.ShapeDtypeStruct((NW, NW, SEGCAP), jnp.float32),
            jax.ShapeDtypeStruct((NW, NW), jnp.int32),
            jax.ShapeDtypeStruct((NW, 16), jnp.float32),
            jax.ShapeDtypeStruct((NW, 16), jnp.float32),
        ),
        mesh=mesh,
        scratch_types=[
            pltpu.VMEM((NW, SEGCAP), jnp.float32),
            pltpu.VMEM((NW, SEGCAP), jnp.float32),
            pltpu.VMEM((NW,), jnp.int32),
            pltpu.VMEM((PIECE,), jnp.float32),
            pltpu.VMEM((PIECE,), jnp.float32),
            pltpu.VMEM((PIECE,), jnp.float32),
            pltpu.VMEM((16,), jnp.float32),
            pltpu.VMEM((16,), jnp.float32),
            pltpu.SemaphoreType.DMA,
        ],
        compiler_params=sc_params,
    )
    sc_group = pl.kernel(
        _sc_group_body,
        out_type=(
            jax.ShapeDtypeStruct((K * CAP,), jnp.float32),
            jax.ShapeDtypeStruct((K * CAP,), jnp.float32),
            jax.ShapeDtypeStruct((K,), jnp.int32),
        ),
        mesh=mesh,
        scratch_types=[
            pltpu.VMEM((BPH * CAP,), jnp.float32),
            pltpu.VMEM((BPH * CAP,), jnp.float32),
            pltpu.VMEM((BPH,), jnp.int32),
            pltpu.VMEM((NW, NW), jnp.int32),
            pltpu.VMEM((SEGCAP,), jnp.float32),
            pltpu.VMEM((SEGCAP,), jnp.float32),
            pltpu.SemaphoreType.DMA,
        ],
        compiler_params=sc_params,
    )
    return sc_part, sc_group


def kernel(risk_scores, times, events):
    sc_part, sc_group = _build_sc_kernels()
    stag_t, stag_sx, cnts, er_p, e_p = sc_part(times, risk_scores, events)
    ts_flat, sx_flat, nb = sc_group(stag_t, stag_sx, cnts)
    ts2 = ts_flat.reshape(K, CAP)
    sx2 = sx_flat.reshape(K, CAP)
    ncol = nb.reshape(K, 1)
    out = _tc_main(ts2, sx2, ncol, er_p, e_p)
    return out[0, 0]


# speedup vs baseline: 1.3800x; 1.0208x over previous
import functools

import jax
import jax.numpy as jnp
from jax import lax
from jax.experimental import pallas as pl
from jax.experimental.pallas import tpu as pltpu
from jax.experimental.pallas import tpu_sc as plsc

N = 1048576
K = 16384
CAP = 128
NC, NS = 2, 16
NW = NC * NS
CHUNK = N // NW
PIECE = 2048
SEGCAP = 1280
BPW = K // NW
BPH = BPW // 2
VECS_B = SEGCAP // 16

def _wid():
    return lax.axis_index("s") * NC + lax.axis_index("c")


def _sc_part_body(t_hbm, r_hbm, e_hbm, stag_t, stag_sx, cnt_out, er_out, e_out,
                  stg_t, stg_sx, cnt32, tbuf, rbuf, ebuf, erbuf, ebuf16, insem):
    wid = _wid()
    base = wid * CHUNK

    def zbody(i, carry):
        cnt32[pl.ds(i * 16, 16)] = jnp.zeros((16,), jnp.int32)
        return carry

    lax.fori_loop(0, 2, zbody, 0)

    def piece_body(p, accs):
        off = base + p * PIECE
        din = [
            pltpu.async_copy(t_hbm.at[pl.ds(off, PIECE)], tbuf, insem),
            pltpu.async_copy(r_hbm.at[pl.ds(off, PIECE)], rbuf, insem),
            pltpu.async_copy(e_hbm.at[pl.ds(off, PIECE)], ebuf, insem),
        ]
        for d in din:
            d.wait()

        def vec_body(v, accs2):
            er_a, e_a = accs2
            t = tbuf[pl.ds(v * 16, 16)]
            r = rbuf[pl.ds(v * 16, 16)]
            e = ebuf[pl.ds(v * 16, 16)]
            b = jnp.minimum((t * float(K)).astype(jnp.int32), K - 1)
            dst = lax.shift_right_logical(b, 9)
            old = plsc.load_gather(cnt32, [dst])
            dupc, last = plsc.scan_count(dst)
            plsc.addupdate_scatter(cnt32, [dst], dupc, mask=last)
            pos = jnp.minimum(old + dupc - 1, SEGCAP - 1)
            ex = jnp.exp(r)
            sx = jnp.where(e > 0.0, ex, -ex)
            plsc.store_scatter(stg_t, [dst, pos], t)
            plsc.store_scatter(stg_sx, [dst, pos], sx)
            return (er_a + e * r, e_a + e)

        return lax.fori_loop(0, PIECE // 16, vec_body, accs)

    zero16f = jnp.zeros((16,), jnp.float32)
    er_acc, e_acc = lax.fori_loop(0, CHUNK // PIECE, piece_body,
                                  (zero16f, zero16f))
    erbuf[...] = er_acc
    ebuf16[...] = e_acc

    descs = []
    for d in range(NW):
        descs.append(pltpu.async_copy(stg_t.at[d], stag_t.at[d, wid], insem))
        descs.append(pltpu.async_copy(stg_sx.at[d], stag_sx.at[d, wid], insem))
    for dd in descs:
        dd.wait()
    pltpu.sync_copy(cnt32, cnt_out.at[wid])
    pltpu.sync_copy(erbuf, er_out.at[wid])
    pltpu.sync_copy(ebuf16, e_out.at[wid])


def _sc_group_body(stag_t, stag_sx, cnt_hbm, ts_out, sx_out, nb_out,
                   ts_tab, sx_tab, ctab, cnt_vm, tb, sb, insem):
    wid = _wid()
    pltpu.sync_copy(cnt_hbm, cnt_vm)

    for h in range(2):
        hbase = wid * BPW + h * BPH

        def zbody(i, carry):
            ctab[pl.ds(i * 16, 16)] = jnp.zeros((16,), jnp.int32)
            return carry

        lax.fori_loop(0, BPH // 16, zbody, 0)

        def src_body(src, carry):
            din = [
                pltpu.async_copy(stag_t.at[wid, src], tb, insem),
                pltpu.async_copy(stag_sx.at[wid, src], sb, insem),
            ]
            for d in din:
                d.wait()
            srcv = jnp.zeros((16,), jnp.int32) + src
            widv = jnp.zeros((16,), jnp.int32) + wid
            cntv = plsc.load_gather(cnt_vm, [srcv, widv])
            cntv = jnp.minimum(cntv, SEGCAP)

            def vec_body(v, carry2):
                t = tb[pl.ds(v * 16, 16)]
                sx = sb[pl.ds(v * 16, 16)]
                gpos = v * 16 + lax.iota(jnp.int32, 16)
                gmask = gpos < cntv
                b = jnp.minimum((t * float(K)).astype(jnp.int32), K - 1)
                bl = b - hbase
                inb = gmask & (bl >= 0) & (bl < BPH)
                blc = jnp.clip(bl, 0, BPH - 1)
                old = plsc.load_gather(ctab, [blc])
                dupc, last = plsc.scan_count(blc, mask=inb)
                plsc.addupdate_scatter(ctab, [blc], dupc, mask=last & inb)
                slot = jnp.minimum(old + dupc - 1, CAP - 1)
                flat = blc * CAP + slot
                plsc.store_scatter(ts_tab, [flat], t, mask=inb)
                plsc.store_scatter(sx_tab, [flat], sx, mask=inb)
                return carry2

            return lax.fori_loop(0, VECS_B, vec_body, carry)

        lax.fori_loop(0, NW, src_body, 0)
        obase = hbase * CAP
        dout = [
            pltpu.async_copy(ts_tab, ts_out.at[pl.ds(obase, BPH * CAP)], insem),
            pltpu.async_copy(sx_tab, sx_out.at[pl.ds(obase, BPH * CAP)], insem),
            pltpu.async_copy(ctab, nb_out.at[pl.ds(hbase, BPH)], insem),
        ]
        for d in dout:
            d.wait()


NB = 256


def _tc_main_body(ts_ref, sx_ref, n_ref, erp_ref, ep_ref, out_ref, acc):
    g = pl.program_id(0)
    nsteps = pl.num_programs(0)

    @pl.when(g == 0)
    def _():
        acc[0] = 0.0
        acc[1] = 0.0
        acc[2] = 0.0

    ts = ts_ref[...]
    sx = sx_ref[...]
    n = n_ref[...]
    lanes = lax.broadcasted_iota(jnp.int32, (NB, CAP), 1)
    valid = lanes < jnp.minimum(n, CAP)
    expr = jnp.where(valid, jnp.abs(sx), 0.0)

    srow = jnp.sum(expr, axis=1, keepdims=True)
    ri = lax.broadcasted_iota(jnp.int32, (NB, NB), 0)
    rj = lax.broadcasted_iota(jnp.int32, (NB, NB), 1)
    tri = (rj > ri).astype(jnp.float32)
    suf = jax.lax.dot_general(tri, srow, (((1,), (0,)), ((), ())),
                              preferred_element_type=jnp.float32)
    t_base = acc[2] + suf

    k = lax.bitcast_convert_type(ts, jnp.int32)
    sj = (2 * k + 1)[:, None, :]
    ioi = lax.broadcasted_iota(jnp.int32, (CAP, CAP), 0)
    ioj = lax.broadcasted_iota(jnp.int32, (CAP, CAP), 1)
    jgti = (ioj > ioi).astype(jnp.int32)[None, :, :]
    si = (2 * k)[:, :, None] + jgti
    mask = sj > si
    w = jnp.sum(jnp.where(mask, expr[:, None, :], 0.0), axis=2)

    c = t_base + w
    e_on = valid & (sx > 0.0)
    contrib = jnp.sum(jnp.where(e_on, jnp.log(c), 0.0))

    y = contrib - acc[1]
    t_new = acc[0] + y
    acc[1] = (t_new - acc[0]) - y
    acc[0] = t_new
    acc[2] = acc[2] + jnp.sum(srow)

    @pl.when(g == nsteps - 1)
    def _():
        er_tot = jnp.sum(erp_ref[...])
        e_tot = jnp.sum(ep_ref[...])
        out_ref[...] = jnp.full((1, 1), (acc[0] - er_tot) / e_tot,
                                dtype=jnp.float32)


_tc_main = pl.pallas_call(
    _tc_main_body,
    grid=(K // NB,),
    in_specs=[
        pl.BlockSpec((NB, CAP), lambda g: (K // NB - 1 - g, 0)),
        pl.BlockSpec((NB, CAP), lambda g: (K // NB - 1 - g, 0)),
        pl.BlockSpec((NB, 1), lambda g: (K // NB - 1 - g, 0)),
        pl.BlockSpec((NW, 16), lambda g: (0, 0)),
        pl.BlockSpec((NW, 16), lambda g: (0, 0)),
    ],
    out_specs=pl.BlockSpec((1, 1), lambda g: (0, 0)),
    out_shape=jax.ShapeDtypeStruct((1, 1), jnp.float32),
    scratch_shapes=[pltpu.SMEM((3,), jnp.float32)],
)


@functools.lru_cache(maxsize=1)
def _build_sc_kernels():
    mesh = plsc.VectorSubcoreMesh(
        core_axis_name="c", subcore_axis_name="s", num_cores=NC, num_subcores=NS
    )
    sc_params = pltpu.CompilerParams(needs_layout_passes=False)
    sc_part = pl.kernel(
        _sc_part_body,
        out_type=(
            jax.ShapeDtypeStruct((NW, NW, SEGCAP), jnp.float32),
            jax.ShapeDtypeStruct((NW, NW, SEGCAP), jnp.float32),
            jax.ShapeDtypeStruct((NW, NW), jnp.int32),
            jax.ShapeDtypeStruct((NW, 16), jnp.float32),
            jax.ShapeDtypeStruct((NW, 16), jnp.float32),
        ),
        mesh=mesh,
        scratch_types=[
            pltpu.VMEM((NW, SEGCAP), jnp.float32),
            pltpu.VMEM((NW, SEGCAP), jnp.float32),
            pltpu.VMEM((NW,), jnp.int32),
            pltpu.VMEM((PIECE,), jnp.float32),
            pltpu.VMEM((PIECE,), jnp.float32),
            pltpu.VMEM((PIECE,), jnp.float32),
            pltpu.VMEM((16,), jnp.float32),
            pltpu.VMEM((16,), jnp.float32),
            pltpu.SemaphoreType.DMA,
        ],
        compiler_params=sc_params,
    )
    sc_group = pl.kernel(
        _sc_group_body,
        out_type=(
            jax.ShapeDtypeStruct((K * CAP,), jnp.float32),
            jax.ShapeDtypeStruct((K * CAP,), jnp.float32),
            jax.ShapeDtypeStruct((K,), jnp.int32),
        ),
        mesh=mesh,
        scratch_types=[
            pltpu.VMEM((BPH * CAP,), jnp.float32),
            pltpu.VMEM((BPH * CAP,), jnp.float32),
            pltpu.VMEM((BPH,), jnp.int32),
            pltpu.VMEM((NW, NW), jnp.int32),
            pltpu.VMEM((SEGCAP,), jnp.float32),
            pltpu.VMEM((SEGCAP,), jnp.float32),
            pltpu.SemaphoreType.DMA,
        ],
        compiler_params=sc_params,
    )
    return sc_part, sc_group


def kernel(risk_scores, times, events):
    sc_part, sc_group = _build_sc_kernels()
    stag_t, stag_sx, cnts, er_p, e_p = sc_part(times, risk_scores, events)
    ts_flat, sx_flat, nb = sc_group(stag_t, stag_sx, cnts)
    ts2 = ts_flat.reshape(K, CAP)
    sx2 = sx_flat.reshape(K, CAP)
    ncol = nb.reshape(K, 1)
    out = _tc_main(ts2, sx2, ncol, er_p, e_p)
    return out[0, 0]


# speedup vs baseline: 1.3979x; 1.0130x over previous
import functools

import jax
import jax.numpy as jnp
from jax import lax
from jax.experimental import pallas as pl
from jax.experimental.pallas import tpu as pltpu
from jax.experimental.pallas import tpu_sc as plsc

N = 1048576
K = 16384
CAP = 128
NC, NS = 2, 16
NW = NC * NS
CHUNK = N // NW
PIECE = 2048
SEGCAP = 1280
BPW = K // NW
BPH = BPW // 2
VECS_B = SEGCAP // 16

def _wid():
    return lax.axis_index("s") * NC + lax.axis_index("c")


def _sc_part_body(t_hbm, r_hbm, e_hbm, stag_t, stag_sx, cnt_out, er_out, e_out,
                  stg_t, stg_sx, cnt32, tbuf, rbuf, ebuf, erbuf, ebuf16, insem):
    wid = _wid()
    base = wid * CHUNK

    def zbody(i, carry):
        cnt32[pl.ds(i * 16, 16)] = jnp.zeros((16,), jnp.int32)
        return carry

    lax.fori_loop(0, 2, zbody, 0)

    def piece_body(p, accs):
        off = base + p * PIECE
        din = [
            pltpu.async_copy(t_hbm.at[pl.ds(off, PIECE)], tbuf, insem),
            pltpu.async_copy(r_hbm.at[pl.ds(off, PIECE)], rbuf, insem),
            pltpu.async_copy(e_hbm.at[pl.ds(off, PIECE)], ebuf, insem),
        ]
        for d in din:
            d.wait()

        def vec_body(v, accs2):
            er_a, e_a = accs2
            t = tbuf[pl.ds(v * 16, 16)]
            r = rbuf[pl.ds(v * 16, 16)]
            e = ebuf[pl.ds(v * 16, 16)]
            b = jnp.minimum((t * float(K)).astype(jnp.int32), K - 1)
            dst = lax.shift_right_logical(b, 9)
            old = plsc.load_gather(cnt32, [dst])
            dupc, last = plsc.scan_count(dst)
            plsc.addupdate_scatter(cnt32, [dst], dupc, mask=last)
            pos = jnp.minimum(old + dupc - 1, SEGCAP - 1)
            ex = jnp.exp(r)
            sx = jnp.where(e > 0.0, ex, -ex)
            plsc.store_scatter(stg_t, [dst, pos], t)
            plsc.store_scatter(stg_sx, [dst, pos], sx)
            return (er_a + e * r, e_a + e)

        return lax.fori_loop(0, PIECE // 16, vec_body, accs)

    zero16f = jnp.zeros((16,), jnp.float32)
    er_acc, e_acc = lax.fori_loop(0, CHUNK // PIECE, piece_body,
                                  (zero16f, zero16f))
    erbuf[...] = er_acc
    ebuf16[...] = e_acc

    descs = []
    for d in range(NW):
        descs.append(pltpu.async_copy(stg_t.at[d], stag_t.at[d, wid], insem))
        descs.append(pltpu.async_copy(stg_sx.at[d], stag_sx.at[d, wid], insem))
    for dd in descs:
        dd.wait()
    pltpu.sync_copy(cnt32, cnt_out.at[wid])
    pltpu.sync_copy(erbuf, er_out.at[wid])
    pltpu.sync_copy(ebuf16, e_out.at[wid])


def _sc_group_body(stag_t, stag_sx, cnt_hbm, ts_out, sx_out, nb_out,
                   ts_tab, sx_tab, ctab, cnt_vm, tb, sb, insem):
    wid = _wid()
    pltpu.sync_copy(cnt_hbm, cnt_vm)

    for h in range(2):
        hbase = wid * BPW + h * BPH

        def zbody(i, carry):
            ctab[pl.ds(i * 16, 16)] = jnp.zeros((16,), jnp.int32)
            return carry

        lax.fori_loop(0, BPH // 16, zbody, 0)

        def src_body(src, carry):
            din = [
                pltpu.async_copy(stag_t.at[wid, src], tb, insem),
                pltpu.async_copy(stag_sx.at[wid, src], sb, insem),
            ]
            for d in din:
                d.wait()
            srcv = jnp.zeros((16,), jnp.int32) + src
            widv = jnp.zeros((16,), jnp.int32) + wid
            cntv = plsc.load_gather(cnt_vm, [srcv, widv])
            cntv = jnp.minimum(cntv, SEGCAP)

            def vec_body(v, carry2):
                t = tb[pl.ds(v * 16, 16)]
                sx = sb[pl.ds(v * 16, 16)]
                gpos = v * 16 + lax.iota(jnp.int32, 16)
                gmask = gpos < cntv
                b = jnp.minimum((t * float(K)).astype(jnp.int32), K - 1)
                bl = b - hbase
                inb = gmask & (bl >= 0) & (bl < BPH)
                blc = jnp.clip(bl, 0, BPH - 1)
                old = plsc.load_gather(ctab, [blc])
                dupc, last = plsc.scan_count(blc, mask=inb)
                plsc.addupdate_scatter(ctab, [blc], dupc, mask=last & inb)
                slot = jnp.minimum(old + dupc - 1, CAP - 1)
                flat = blc * CAP + slot
                plsc.store_scatter(ts_tab, [flat], t, mask=inb)
                plsc.store_scatter(sx_tab, [flat], sx, mask=inb)
                return carry2

            return lax.fori_loop(0, VECS_B, vec_body, carry)

        lax.fori_loop(0, NW, src_body, 0)
        obase = hbase * CAP
        dout = [
            pltpu.async_copy(ts_tab, ts_out.at[pl.ds(obase, BPH * CAP)], insem),
            pltpu.async_copy(sx_tab, sx_out.at[pl.ds(obase, BPH * CAP)], insem),
            pltpu.async_copy(ctab, nb_out.at[pl.ds(hbase, BPH)], insem),
        ]
        for d in dout:
            d.wait()


NB = 512


def _tc_main_body(ts_ref, sx_ref, n_ref, erp_ref, ep_ref, out_ref, acc):
    g = pl.program_id(0)
    nsteps = pl.num_programs(0)

    @pl.when(g == 0)
    def _():
        acc[0] = 0.0
        acc[1] = 0.0
        acc[2] = 0.0

    ts = ts_ref[...]
    sx = sx_ref[...]
    n = n_ref[...]
    lanes = lax.broadcasted_iota(jnp.int32, (NB, CAP), 1)
    valid = lanes < jnp.minimum(n, CAP)
    expr = jnp.where(valid, jnp.abs(sx), 0.0)

    srow = jnp.sum(expr, axis=1, keepdims=True)
    ri = lax.broadcasted_iota(jnp.int32, (NB, NB), 0)
    rj = lax.broadcasted_iota(jnp.int32, (NB, NB), 1)
    tri = (rj > ri).astype(jnp.float32)
    suf = jax.lax.dot_general(tri, srow, (((1,), (0,)), ((), ())),
                              preferred_element_type=jnp.float32)
    t_base = acc[2] + suf

    k = lax.bitcast_convert_type(ts, jnp.int32)
    sj = (2 * k + 1)[:, None, :]
    ioi = lax.broadcasted_iota(jnp.int32, (CAP, CAP), 0)
    ioj = lax.broadcasted_iota(jnp.int32, (CAP, CAP), 1)
    jgti = (ioj > ioi).astype(jnp.int32)[None, :, :]
    si = (2 * k)[:, :, None] + jgti
    mask = sj > si
    w = jnp.sum(jnp.where(mask, expr[:, None, :], 0.0), axis=2)

    c = t_base + w
    e_on = valid & (sx > 0.0)
    contrib = jnp.sum(jnp.where(e_on, jnp.log(c), 0.0))

    y = contrib - acc[1]
    t_new = acc[0] + y
    acc[1] = (t_new - acc[0]) - y
    acc[0] = t_new
    acc[2] = acc[2] + jnp.sum(srow)

    @pl.when(g == nsteps - 1)
    def _():
        er_tot = jnp.sum(erp_ref[...])
        e_tot = jnp.sum(ep_ref[...])
        out_ref[...] = jnp.full((1, 1), (acc[0] - er_tot) / e_tot,
                                dtype=jnp.float32)


_tc_main = pl.pallas_call(
    _tc_main_body,
    grid=(K // NB,),
    in_specs=[
        pl.BlockSpec((NB, CAP), lambda g: (K // NB - 1 - g, 0)),
        pl.BlockSpec((NB, CAP), lambda g: (K // NB - 1 - g, 0)),
        pl.BlockSpec((NB, 1), lambda g: (K // NB - 1 - g, 0)),
        pl.BlockSpec((NW, 16), lambda g: (0, 0)),
        pl.BlockSpec((NW, 16), lambda g: (0, 0)),
    ],
    out_specs=pl.BlockSpec((1, 1), lambda g: (0, 0)),
    out_shape=jax.ShapeDtypeStruct((1, 1), jnp.float32),
    scratch_shapes=[pltpu.SMEM((3,), jnp.float32)],
)


@functools.lru_cache(maxsize=1)
def _build_sc_kernels():
    mesh = plsc.VectorSubcoreMesh(
        core_axis_name="c", subcore_axis_name="s", num_cores=NC, num_subcores=NS
    )
    sc_params = pltpu.CompilerParams(needs_layout_passes=False)
    sc_part = pl.kernel(
        _sc_part_body,
        out_type=(
            jax.ShapeDtypeStruct((NW, NW, SEGCAP), jnp.float32),
            jax.ShapeDtypeStruct((NW, NW, SEGCAP), jnp.float32),
            jax.ShapeDtypeStruct((NW, NW), jnp.int32),
            jax.ShapeDtypeStruct((NW, 16), jnp.float32),
            jax.ShapeDtypeStruct((NW, 16), jnp.float32),
        ),
        mesh=mesh,
        scratch_types=[
            pltpu.VMEM((NW, SEGCAP), jnp.float32),
            pltpu.VMEM((NW, SEGCAP), jnp.float32),
            pltpu.VMEM((NW,), jnp.int32),
            pltpu.VMEM((PIECE,), jnp.float32),
            pltpu.VMEM((PIECE,), jnp.float32),
            pltpu.VMEM((PIECE,), jnp.float32),
            pltpu.VMEM((16,), jnp.float32),
            pltpu.VMEM((16,), jnp.float32),
            pltpu.SemaphoreType.DMA,
        ],
        compiler_params=sc_params,
    )
    sc_group = pl.kernel(
        _sc_group_body,
        out_type=(
            jax.ShapeDtypeStruct((K * CAP,), jnp.float32),
            jax.ShapeDtypeStruct((K * CAP,), jnp.float32),
            jax.ShapeDtypeStruct((K,), jnp.int32),
        ),
        mesh=mesh,
        scratch_types=[
            pltpu.VMEM((BPH * CAP,), jnp.float32),
            pltpu.VMEM((BPH * CAP,), jnp.float32),
            pltpu.VMEM((BPH,), jnp.int32),
            pltpu.VMEM((NW, NW), jnp.int32),
            pltpu.VMEM((SEGCAP,), jnp.float32),
            pltpu.VMEM((SEGCAP,), jnp.float32),
            pltpu.SemaphoreType.DMA,
        ],
        compiler_params=sc_params,
    )
    return sc_part, sc_group


def kernel(risk_scores, times, events):
    sc_part, sc_group = _build_sc_kernels()
    stag_t, stag_sx, cnts, er_p, e_p = sc_part(times, risk_scores, events)
    ts_flat, sx_flat, nb = sc_group(stag_t, stag_sx, cnts)
    ts2 = ts_flat.reshape(K, CAP)
    sx2 = sx_flat.reshape(K, CAP)
    ncol = nb.reshape(K, 1)
    out = _tc_main(ts2, sx2, ncol, er_p, e_p)
    return out[0, 0]
